# baseline (device time: 125064 ns/iter reference)
import jax
import jax.numpy as jnp
from jax import lax
from jax.experimental import pallas as pl
from jax.experimental.pallas import tpu as pltpu

N_SC = 32
N_EC = 8


def kernel(x):
    m, n = x.shape
    n_out = n // 2
    half = m // 2
    r_sc = half // N_SC
    r_ec = m // N_EC

    def body(x_hbm, out_hbm, sload_buf, send_buf, recv_buf, eload_buf,
             local_vmem, sload_sems, eload_sems, store_sem, fstore_sems,
             send_x_sems, recv_x_sems, send_y_sems, recv_y_sems):
        my_x = lax.axis_index("x")
        my_y = lax.axis_index("y")
        x_peer = (1 - my_x, my_y)
        y_peer = (my_x, 1 - my_y)

        peer_c0 = (1 - my_x) * n_out
        my_c0 = my_x * n_out
        send_r0 = my_y * half
        in_x_base = (1 - my_x) * m + my_y * half
        in_y_base = (1 - my_x) * m + (1 - my_y) * half

        barrier_sem = pltpu.get_barrier_semaphore()
        for nbr in (x_peer, y_peer):
            pl.semaphore_signal(
                barrier_sem, inc=1,
                device_id=nbr, device_id_type=pl.DeviceIdType.MESH,
            )
        pl.semaphore_wait(barrier_sem, 2)

        sloads = []
        for c in range(N_SC):
            d = pltpu.make_async_copy(
                x_hbm.at[pl.ds(send_r0 + c * r_sc, r_sc),
                         pl.ds(peer_c0, n_out)],
                sload_buf.at[c], sload_sems.at[c])
            d.start()
            sloads.append(d)
        rdma_x = []
        for c in range(N_SC):
            sloads[c].wait()
            send_buf[c] = sload_buf[c].astype(jnp.bfloat16)
            rx = pltpu.make_async_remote_copy(
                src_ref=send_buf.at[c],
                dst_ref=recv_buf.at[c],
                send_sem=send_x_sems.at[c],
                recv_sem=recv_x_sems.at[c],
                device_id=x_peer,
                device_id_type=pl.DeviceIdType.MESH,
            )
            rx.start()
            rdma_x.append(rx)

        def e_load(c):
            d = pltpu.make_async_copy(
                x_hbm.at[pl.ds(c * r_ec, r_ec), pl.ds(my_c0, n_out)],
                eload_buf.at[c % 2], eload_sems.at[c % 2])
            d.start()
            return d

        e_cur = e_load(0)

        def e_step(c):
            nonlocal e_cur
            nxt = e_load(c + 1) if c + 1 < N_EC else None
            e_cur.wait()
            local_vmem[pl.ds(c * r_ec, r_ec), :] = eload_buf[
                c % 2].astype(jnp.bfloat16)
            e_cur = nxt

        rdma_y = []
        fstores = []
        for c in range(N_SC):
            rows = pl.ds(in_x_base + c * r_sc, r_sc)
            rdma_x[c].wait_recv()
            ry = pltpu.make_async_remote_copy(
                src_ref=recv_buf.at[c],
                dst_ref=out_hbm.at[rows],
                send_sem=send_y_sems.at[c],
                recv_sem=recv_y_sems.at[c],
                device_id=y_peer,
                device_id_type=pl.DeviceIdType.MESH,
            )
            ry.start()
            rdma_y.append(ry)
            fs = pltpu.make_async_copy(
                recv_buf.at[c], out_hbm.at[rows], fstore_sems.at[c])
            fs.start()
            fstores.append(fs)
            if c % (N_SC // N_EC) == 0:
                e_step(c // (N_SC // N_EC))

        store = pltpu.make_async_copy(
            local_vmem, out_hbm.at[pl.ds(my_x * m, m)], store_sem)
        store.start()

        for c in range(N_SC):
            rows = pl.ds(in_y_base + c * r_sc, r_sc)
            ywait = pltpu.make_async_remote_copy(
                src_ref=recv_buf.at[c],
                dst_ref=out_hbm.at[rows],
                send_sem=send_y_sems.at[c],
                recv_sem=recv_y_sems.at[c],
                device_id=y_peer,
                device_id_type=pl.DeviceIdType.MESH,
            )
            ywait.wait_recv()
        for rx in rdma_x:
            rx.wait_send()
        for ry in rdma_y:
            ry.wait_send()
        for fs in fstores:
            fs.wait()
        store.wait()

    return pl.pallas_call(
        body,
        out_shape=jax.ShapeDtypeStruct((2 * m, n_out), jnp.bfloat16),
        in_specs=[pl.BlockSpec(memory_space=pl.ANY)],
        out_specs=pl.BlockSpec(memory_space=pl.ANY),
        scratch_shapes=[
            pltpu.VMEM((N_SC, r_sc, n_out), jnp.float32),
            pltpu.VMEM((N_SC, r_sc, n_out), jnp.bfloat16),
            pltpu.VMEM((N_SC, r_sc, n_out), jnp.bfloat16),
            pltpu.VMEM((2, r_ec, n_out), jnp.float32),
            pltpu.VMEM((m, n_out), jnp.bfloat16),
            pltpu.SemaphoreType.DMA((N_SC,)),
            pltpu.SemaphoreType.DMA((2,)),
            pltpu.SemaphoreType.DMA,
            pltpu.SemaphoreType.DMA((N_SC,)),
            pltpu.SemaphoreType.DMA((N_SC,)),
            pltpu.SemaphoreType.DMA((N_SC,)),
            pltpu.SemaphoreType.DMA((N_SC,)),
            pltpu.SemaphoreType.DMA((N_SC,)),
        ],
        compiler_params=pltpu.CompilerParams(
            collective_id=0, vmem_limit_bytes=80 * 1024 * 1024),
    )(x)
